# baseline (speedup 1.0000x reference)
import functools

import jax
import jax.numpy as jnp
from jax import lax
from jax.experimental import pallas as pl
from jax.experimental.pallas import tpu as pltpu
from jax.experimental.pallas import tpu_sc as plsc

N = 50000
E = 800000
NG = 512
IN_C = 4
H = 64
OUT_C = 3

NPAD = 51200
HALF = NPAD // 2
ACC_ROWS = HALF + 32
TRASH = HALF
NSUB = 16
CH = 128
NCH = 392
EPT = NCH * CH
EPAD = NSUB * EPT
BR = 2048
GRID = NPAD // BR
INV_BN = 0.9999950000374996


@functools.lru_cache(maxsize=None)
def _sc_agg(D):
    mesh = plsc.VectorSubcoreMesh(
        core_axis_name="c", subcore_axis_name="s", num_cores=2, num_subcores=NSUB
    )
    nk = D // 16

    @functools.partial(
        pl.kernel,
        out_type=jax.ShapeDtypeStruct((NPAD, D), jnp.float32),
        mesh=mesh,
        compiler_params=pltpu.CompilerParams(use_tc_tiling_on_sc=False),
        scratch_types=[
            pltpu.VMEM_SHARED((ACC_ROWS, D), jnp.float32),
            pltpu.VMEM((CH,), jnp.int32),
            pltpu.VMEM((CH,), jnp.int32),
            pltpu.VMEM((CH,), jnp.int32),
            pltpu.VMEM((CH,), jnp.int32),
            pltpu.VMEM((CH,), jnp.int32),
            pltpu.VMEM((CH,), jnp.int32),
            pltpu.VMEM((CH,), jnp.float32),
            pltpu.VMEM((CH,), jnp.float32),
            pltpu.VMEM((CH, D), jnp.float32),
            pltpu.VMEM((CH, D), jnp.float32),
            pltpu.VMEM((CH,), jnp.int32),
            pltpu.VMEM((CH,), jnp.int32),
            pltpu.VMEM((2, D), jnp.float32),
            pltpu.SemaphoreType.DMA,
            pltpu.SemaphoreType.DMA,
            pltpu.SemaphoreType.DMA,
        ],
    )
    def k(x_hbm, src_hbm, dst_hbm, attr_hbm, wb_hbm, out_hbm,
          acc, src0, src1, dst0, dst1, loc0, loc1, attr0, attr1,
          rows0, rows1, gidx0, gidx1, wb_v, isem, gsem, ssem):
        c = lax.axis_index("c")
        s = lax.axis_index("s")
        srcb = (src0, src1)
        dstb = (dst0, dst1)
        locb = (loc0, loc1)
        attrb = (attr0, attr1)
        rowsb = (rows0, rows1)
        gidxb = (gidx0, gidx1)

        pltpu.sync_copy(wb_hbm, wb_v)
        w_k = [wb_v[0, pl.ds(kk * 16, 16)] for kk in range(nk)]
        b_k = [wb_v[1, pl.ds(kk * 16, 16)] for kk in range(nk)]

        zero16 = jnp.zeros((16,), jnp.float32)

        def zrow(r, carry):
            for kk in range(nk):
                rows0[r, pl.ds(kk * 16, 16)] = zero16
            return carry

        lax.fori_loop(0, CH, zrow, 0)
        stripe = ACC_ROWS // NSUB
        for i in range(stripe // CH):
            pltpu.sync_copy(rows0, acc.at[pl.ds(s * stripe + i * CH, CH)])
        rem = stripe % CH
        if rem:
            pltpu.sync_copy(rows0.at[pl.ds(0, rem)],
                            acc.at[pl.ds(s * stripe + (stripe // CH) * CH, rem)])
        plsc.subcore_barrier()

        base = c * HALF
        ebase = s * EPT

        def idx_start(j, b):
            eb = ebase + j * CH
            pltpu.async_copy(src_hbm.at[pl.ds(eb, CH)], srcb[b], isem)
            pltpu.async_copy(dst_hbm.at[pl.ds(eb, CH)], dstb[b], isem)
            pltpu.async_copy(attr_hbm.at[pl.ds(eb, CH)], attrb[b], isem)

        def idx_wait(j, b):
            eb = ebase + j * CH
            pltpu.make_async_copy(src_hbm.at[pl.ds(eb, CH)], srcb[b], isem).wait()
            pltpu.make_async_copy(dst_hbm.at[pl.ds(eb, CH)], dstb[b], isem).wait()
            pltpu.make_async_copy(attr_hbm.at[pl.ds(eb, CH)], attrb[b], isem).wait()

        def drain_scatter(b):
            pltpu.make_async_copy(rowsb[b], acc.at[locb[b]], ssem).wait()

        def mask_gidx(b):
            for i in range(CH // 16):
                d = dstb[b][pl.ds(i * 16, 16)]
                local = d - base
                ok = (local >= 0) & (local < HALF)
                gidxb[b][pl.ds(i * 16, 16)] = jnp.where(
                    ok, srcb[b][pl.ds(i * 16, 16)], 0)

        def step(j, b, start_gather=True, start_idx=True, first=False):
            if start_gather:
                if not first:
                    drain_scatter(1 - b)
                idx_wait(j + 1, 1 - b)
                mask_gidx(1 - b)
                pltpu.async_copy(x_hbm.at[gidxb[1 - b]], rowsb[1 - b], gsem)
            for i in range(CH // 16):
                d = dstb[b][pl.ds(i * 16, 16)]
                local = d - base
                ok = (local >= 0) & (local < HALF)
                locb[b][pl.ds(i * 16, 16)] = jnp.where(ok, local, TRASH)
            pltpu.make_async_copy(x_hbm.at[gidxb[b]], rowsb[b], gsem).wait()

            def group(g):
                a16 = attrb[b][pl.ds(g * 16, 16)]
                for el in range(16):
                    av = lax.gather(
                        a16, jnp.full((16, 1), el, jnp.int32),
                        lax.GatherDimensionNumbers(
                            offset_dims=(), collapsed_slice_dims=(0,),
                            start_index_map=(0,)),
                        slice_sizes=(1,),
                        mode=lax.GatherScatterMode.PROMISE_IN_BOUNDS)
                    e = g * 16 + el
                    for kk in range(nk):
                        r = rowsb[b][e, pl.ds(kk * 16, 16)]
                        m = jnp.maximum(r + (av * w_k[kk] + b_k[kk]), 0.0)
                        rowsb[b][e, pl.ds(kk * 16, 16)] = m

            plsc.parallel_loop(0, CH // 16, 1, unroll=2, carry=None)(group)
            if start_idx:
                idx_start(j + 2, b)
            pltpu.async_copy(rowsb[b], acc.at[locb[b]], ssem, add=True)

        idx_start(0, 0)
        idx_start(1, 1)
        idx_wait(0, 0)
        mask_gidx(0)
        pltpu.async_copy(x_hbm.at[gidx0], rows0, gsem)

        step(0, 0, first=True)
        step(1, 1)

        def pair(p, carry):
            j = 2 * p
            step(j, 0)
            step(j + 1, 1)
            return carry

        lax.fori_loop(1, NCH // 2 - 1, pair, 0)
        step(NCH - 2, 0, start_idx=False)
        step(NCH - 1, 1, start_gather=False, start_idx=False)
        drain_scatter(0)
        drain_scatter(1)
        plsc.subcore_barrier()

        oper = HALF // NSUB
        for i in range(oper // CH):
            pltpu.sync_copy(acc.at[pl.ds(s * oper + i * CH, CH)], rows0)
            pltpu.sync_copy(rows0, out_hbm.at[pl.ds(base + s * oper + i * CH, CH)])
        orem = oper % CH
        if orem:
            ob = (oper // CH) * CH
            pltpu.sync_copy(acc.at[pl.ds(s * oper + ob, orem)],
                            rows0.at[pl.ds(0, orem)])
            pltpu.sync_copy(rows0.at[pl.ds(0, orem)],
                            out_hbm.at[pl.ds(base + s * oper + ob, orem)])

    return k


@functools.lru_cache(maxsize=None)
def _mlp(D):
    def body(eps_ref, x_ref, agg_ref, w1_ref, w2_ref, aux_ref, o_ref):
        eps = eps_ref[0]
        h = (1.0 + eps) * x_ref[...] + agg_ref[...]
        t = lax.dot_general(h, w1_ref[...], (((1,), (0,)), ((), ())),
                            preferred_element_type=jnp.float32)
        t = jnp.maximum(t + aux_ref[0:1, :], 0.0)
        u = lax.dot_general(t, w2_ref[...], (((1,), (0,)), ((), ())),
                            preferred_element_type=jnp.float32)
        u = u + aux_ref[1:2, :]
        u = u * (aux_ref[2:3, :] * INV_BN) + aux_ref[3:4, :]
        o_ref[...] = jnp.maximum(u, 0.0)

    return pl.pallas_call(
        body,
        grid=(GRID,),
        in_specs=[
            pl.BlockSpec(memory_space=pltpu.SMEM),
            pl.BlockSpec((BR, D), lambda i: (i, 0)),
            pl.BlockSpec((BR, D), lambda i: (i, 0)),
            pl.BlockSpec((D, H), lambda i: (0, 0)),
            pl.BlockSpec((H, H), lambda i: (0, 0)),
            pl.BlockSpec((4, H), lambda i: (0, 0)),
        ],
        out_specs=pl.BlockSpec((BR, H), lambda i: (i, 0)),
        out_shape=jax.ShapeDtypeStruct((NPAD, H), jnp.float32),
    )


def _pool():
    def body(h_ref, b_ref, wc1_ref, bc1_ref, wc2_ref, bc2_ref, o_ref, acc):
        j = pl.program_id(0)

        @pl.when(j == 0)
        def _init():
            acc[...] = jnp.zeros_like(acc)

        bid = b_ref[0, 0, :]
        onehot = (bid[:, None] == lax.broadcasted_iota(jnp.int32, (BR, NG), 1)
                  ).astype(jnp.float32)
        hext = jnp.concatenate(
            [h_ref[...], jnp.ones((BR, H), jnp.float32)], axis=1)
        acc[...] += lax.dot_general(onehot, hext, (((0,), (0,)), ((), ())),
                                    preferred_element_type=jnp.float32)

        @pl.when(j == GRID - 1)
        def _fin():
            a = acc[...]
            pooled = a[:, :H] / jnp.maximum(a[:, H:H + 1], 1.0)
            z = lax.dot_general(pooled, wc1_ref[...], (((1,), (0,)), ((), ())),
                                preferred_element_type=jnp.float32)
            z = jnp.maximum(z + bc1_ref[...], 0.0)
            o = lax.dot_general(z, wc2_ref[...], (((1,), (0,)), ((), ())),
                                preferred_element_type=jnp.float32)
            o_ref[...] = o + bc2_ref[...]

    return pl.pallas_call(
        body,
        grid=(GRID,),
        in_specs=[
            pl.BlockSpec((BR, H), lambda i: (i, 0)),
            pl.BlockSpec((1, 1, BR), lambda i: (i, 0, 0)),
            pl.BlockSpec((H, H), lambda i: (0, 0)),
            pl.BlockSpec((1, H), lambda i: (0, 0)),
            pl.BlockSpec((H, 128), lambda i: (0, 0)),
            pl.BlockSpec((1, 128), lambda i: (0, 0)),
        ],
        out_specs=pl.BlockSpec((NG, 128), lambda i: (0, 0)),
        out_shape=jax.ShapeDtypeStruct((NG, 128), jnp.float32),
        scratch_shapes=[pltpu.VMEM((NG, 128), jnp.float32)],
    )


def kernel(x, edge_index, batch, edge_attr, params):
    f32 = jnp.float32
    src = jnp.concatenate([edge_index[0], jnp.zeros((EPAD - E,), jnp.int32)])
    dst = jnp.concatenate([edge_index[1], jnp.full((EPAD - E,), N, jnp.int32)])
    attr = jnp.concatenate([edge_attr[:, 0], jnp.zeros((EPAD - E,), f32)])
    batch_p = jnp.concatenate(
        [batch, jnp.full((NPAD - N,), NG, jnp.int32)]).reshape(GRID, 1, BR)

    h = jnp.zeros((NPAD, 16), f32).at[:N, :IN_C].set(x)
    for li, lp in enumerate(params["layers"]):
        D = 16 if li == 0 else H
        w = lp["edge_lin"]["W"][0]
        b = lp["edge_lin"]["b"]
        W1 = lp["nn1"]["W"]
        if li == 0:
            w = jnp.pad(w, (0, 16 - IN_C))
            b = jnp.pad(b, (0, 16 - IN_C))
            W1 = jnp.pad(W1, ((0, 16 - IN_C), (0, 0)))
        wb = jnp.stack([w, b])
        agg = _sc_agg(D)(h, src, dst, attr, wb)
        aux = jnp.stack(
            [lp["nn1"]["b"], lp["nn2"]["b"], lp["bn_gamma"], lp["bn_beta"]])
        h = _mlp(D)(jnp.reshape(lp["eps"], (1,)), h, agg, W1, lp["nn2"]["W"], aux)

    wc2 = jnp.pad(params["cls"]["l2"]["W"], ((0, 0), (0, 128 - OUT_C)))
    bc2 = jnp.pad(params["cls"]["l2"]["b"], (0, 128 - OUT_C)).reshape(1, 128)
    out = _pool()(h, batch_p, params["cls"]["l1"]["W"],
                  params["cls"]["l1"]["b"].reshape(1, H), wc2, bc2)
    return out[:, :OUT_C]


# speedup vs baseline: 19.9274x; 19.9274x over previous
import functools

import jax
import jax.numpy as jnp
from jax import lax
from jax.experimental import pallas as pl
from jax.experimental.pallas import tpu as pltpu
from jax.experimental.pallas import tpu_sc as plsc

N = 50000
E = 800000
NG = 512
IN_C = 4
H = 64
OUT_C = 3

NPAD = 51200
HALF = NPAD // 2
ACC_ROWS = HALF + 256
TRASH = HALF
NSUB = 16
CH = 128
NCH = 392
EPT = NCH * CH
EPAD = NSUB * EPT
BR = 2048
GRID = NPAD // BR
INV_BN = 0.9999950000374996


@functools.lru_cache(maxsize=None)
def _sc_agg(D):
    mesh = plsc.VectorSubcoreMesh(
        core_axis_name="c", subcore_axis_name="s", num_cores=2, num_subcores=NSUB
    )
    nk = D // 16

    @functools.partial(
        pl.kernel,
        out_type=jax.ShapeDtypeStruct((NPAD, D), jnp.float32),
        mesh=mesh,
        compiler_params=pltpu.CompilerParams(use_tc_tiling_on_sc=False),
        scratch_types=[
            pltpu.VMEM_SHARED((ACC_ROWS, D), jnp.float32),
            pltpu.VMEM((CH,), jnp.int32),
            pltpu.VMEM((CH,), jnp.int32),
            pltpu.VMEM((CH,), jnp.int32),
            pltpu.VMEM((CH,), jnp.int32),
            pltpu.VMEM((CH,), jnp.int32),
            pltpu.VMEM((CH,), jnp.int32),
            pltpu.VMEM((CH,), jnp.float32),
            pltpu.VMEM((CH,), jnp.float32),
            pltpu.VMEM((CH, D), jnp.float32),
            pltpu.VMEM((CH, D), jnp.float32),
            pltpu.VMEM((2, D), jnp.float32),
            pltpu.SemaphoreType.DMA,
            pltpu.SemaphoreType.DMA,
            pltpu.SemaphoreType.DMA,
        ],
    )
    def k(x_hbm, src_hbm, dst_hbm, attr_hbm, wb_hbm, out_hbm,
          acc, src0, src1, dst0, dst1, loc0, loc1, attr0, attr1,
          rows0, rows1, wb_v, isem, gsem, ssem):
        c = lax.axis_index("c")
        s = lax.axis_index("s")
        srcb = (src0, src1)
        dstb = (dst0, dst1)
        locb = (loc0, loc1)
        attrb = (attr0, attr1)
        rowsb = (rows0, rows1)

        pltpu.sync_copy(wb_hbm, wb_v)
        w_k = [wb_v[0, pl.ds(kk * 16, 16)] for kk in range(nk)]
        b_k = [wb_v[1, pl.ds(kk * 16, 16)] for kk in range(nk)]

        zero16 = jnp.zeros((16,), jnp.float32)
        lanes = lax.broadcasted_iota(jnp.int32, (16,), 0)
        trashv = TRASH + s * 16 + lanes

        def zrow(r, carry):
            for kk in range(nk):
                rows0[r, pl.ds(kk * 16, 16)] = zero16
            return carry

        lax.fori_loop(0, CH, zrow, 0)
        stripe = ACC_ROWS // NSUB
        for i in range(stripe // CH):
            pltpu.sync_copy(rows0, acc.at[pl.ds(s * stripe + i * CH, CH)])
        rem = stripe % CH
        if rem:
            pltpu.sync_copy(rows0.at[pl.ds(0, rem)],
                            acc.at[pl.ds(s * stripe + (stripe // CH) * CH, rem)])
        plsc.subcore_barrier()

        base = c * HALF
        ebase = s * EPT

        def idx_start(j, b):
            eb = ebase + j * CH
            pltpu.async_copy(src_hbm.at[pl.ds(eb, CH)], srcb[b], isem)
            pltpu.async_copy(dst_hbm.at[pl.ds(eb, CH)], dstb[b], isem)
            pltpu.async_copy(attr_hbm.at[pl.ds(eb, CH)], attrb[b], isem)

        def idx_wait(j, b):
            eb = ebase + j * CH
            pltpu.make_async_copy(src_hbm.at[pl.ds(eb, CH)], srcb[b], isem).wait()
            pltpu.make_async_copy(dst_hbm.at[pl.ds(eb, CH)], dstb[b], isem).wait()
            pltpu.make_async_copy(attr_hbm.at[pl.ds(eb, CH)], attrb[b], isem).wait()

        def drain_scatter(b):
            pltpu.make_async_copy(rowsb[b], acc.at[locb[b]], ssem).wait()

        def step(j, b, start_gather=True, start_idx=True, first=False):
            if start_gather:
                if not first:
                    drain_scatter(1 - b)
                idx_wait(j + 1, 1 - b)
                pltpu.async_copy(x_hbm.at[srcb[1 - b]], rowsb[1 - b], gsem)
            for i in range(CH // 16):
                d = dstb[b][pl.ds(i * 16, 16)]
                local = d - base
                ok = (local >= 0) & (local < HALF)
                locb[b][pl.ds(i * 16, 16)] = jnp.where(ok, local, trashv)
            pltpu.make_async_copy(x_hbm.at[srcb[b]], rowsb[b], gsem).wait()

            def group(g):
                a16 = attrb[b][pl.ds(g * 16, 16)]
                for el in range(16):
                    av = lax.gather(
                        a16, jnp.full((16, 1), el, jnp.int32),
                        lax.GatherDimensionNumbers(
                            offset_dims=(), collapsed_slice_dims=(0,),
                            start_index_map=(0,)),
                        slice_sizes=(1,),
                        mode=lax.GatherScatterMode.PROMISE_IN_BOUNDS)
                    e = g * 16 + el
                    for kk in range(nk):
                        r = rowsb[b][e, pl.ds(kk * 16, 16)]
                        m = jnp.maximum(r + (av * w_k[kk] + b_k[kk]), 0.0)
                        rowsb[b][e, pl.ds(kk * 16, 16)] = m

            plsc.parallel_loop(0, CH // 16, 1, unroll=2, carry=None)(group)
            if start_idx:
                idx_start(j + 2, b)
            pltpu.async_copy(rowsb[b], acc.at[locb[b]], ssem, add=True)

        idx_start(0, 0)
        idx_start(1, 1)
        idx_wait(0, 0)
        pltpu.async_copy(x_hbm.at[src0], rows0, gsem)

        step(0, 0, first=True)
        step(1, 1)

        def pair(p, carry):
            j = 2 * p
            step(j, 0)
            step(j + 1, 1)
            return carry

        lax.fori_loop(1, NCH // 2 - 1, pair, 0)
        step(NCH - 2, 0, start_idx=False)
        step(NCH - 1, 1, start_gather=False, start_idx=False)
        drain_scatter(0)
        drain_scatter(1)
        plsc.subcore_barrier()

        oper = HALF // NSUB
        for i in range(oper // CH):
            pltpu.sync_copy(acc.at[pl.ds(s * oper + i * CH, CH)], rows0)
            pltpu.sync_copy(rows0, out_hbm.at[pl.ds(base + s * oper + i * CH, CH)])
        orem = oper % CH
        if orem:
            ob = (oper // CH) * CH
            pltpu.sync_copy(acc.at[pl.ds(s * oper + ob, orem)],
                            rows0.at[pl.ds(0, orem)])
            pltpu.sync_copy(rows0.at[pl.ds(0, orem)],
                            out_hbm.at[pl.ds(base + s * oper + ob, orem)])

    return k


@functools.lru_cache(maxsize=None)
def _mlp(D):
    def body(eps_ref, x_ref, agg_ref, w1_ref, w2_ref, aux_ref, o_ref):
        eps = eps_ref[0]
        h = (1.0 + eps) * x_ref[...] + agg_ref[...]
        t = lax.dot_general(h, w1_ref[...], (((1,), (0,)), ((), ())),
                            preferred_element_type=jnp.float32)
        t = jnp.maximum(t + aux_ref[0:1, :], 0.0)
        u = lax.dot_general(t, w2_ref[...], (((1,), (0,)), ((), ())),
                            preferred_element_type=jnp.float32)
        u = u + aux_ref[1:2, :]
        u = u * (aux_ref[2:3, :] * INV_BN) + aux_ref[3:4, :]
        o_ref[...] = jnp.maximum(u, 0.0)

    return pl.pallas_call(
        body,
        grid=(GRID,),
        in_specs=[
            pl.BlockSpec(memory_space=pltpu.SMEM),
            pl.BlockSpec((BR, D), lambda i: (i, 0)),
            pl.BlockSpec((BR, D), lambda i: (i, 0)),
            pl.BlockSpec((D, H), lambda i: (0, 0)),
            pl.BlockSpec((H, H), lambda i: (0, 0)),
            pl.BlockSpec((4, H), lambda i: (0, 0)),
        ],
        out_specs=pl.BlockSpec((BR, H), lambda i: (i, 0)),
        out_shape=jax.ShapeDtypeStruct((NPAD, H), jnp.float32),
    )


def _pool():
    def body(h_ref, b_ref, wc1_ref, bc1_ref, wc2_ref, bc2_ref, o_ref, acc):
        j = pl.program_id(0)

        @pl.when(j == 0)
        def _init():
            acc[...] = jnp.zeros_like(acc)

        bid = b_ref[0, 0, :]
        onehot = (bid[:, None] == lax.broadcasted_iota(jnp.int32, (BR, NG), 1)
                  ).astype(jnp.float32)
        hext = jnp.concatenate(
            [h_ref[...], jnp.ones((BR, H), jnp.float32)], axis=1)
        acc[...] += lax.dot_general(onehot, hext, (((0,), (0,)), ((), ())),
                                    preferred_element_type=jnp.float32)

        @pl.when(j == GRID - 1)
        def _fin():
            a = acc[...]
            pooled = a[:, :H] / jnp.maximum(a[:, H:H + 1], 1.0)
            z = lax.dot_general(pooled, wc1_ref[...], (((1,), (0,)), ((), ())),
                                preferred_element_type=jnp.float32)
            z = jnp.maximum(z + bc1_ref[...], 0.0)
            o = lax.dot_general(z, wc2_ref[...], (((1,), (0,)), ((), ())),
                                preferred_element_type=jnp.float32)
            o_ref[...] = o + bc2_ref[...]

    return pl.pallas_call(
        body,
        grid=(GRID,),
        in_specs=[
            pl.BlockSpec((BR, H), lambda i: (i, 0)),
            pl.BlockSpec((1, 1, BR), lambda i: (i, 0, 0)),
            pl.BlockSpec((H, H), lambda i: (0, 0)),
            pl.BlockSpec((1, H), lambda i: (0, 0)),
            pl.BlockSpec((H, 128), lambda i: (0, 0)),
            pl.BlockSpec((1, 128), lambda i: (0, 0)),
        ],
        out_specs=pl.BlockSpec((NG, 128), lambda i: (0, 0)),
        out_shape=jax.ShapeDtypeStruct((NG, 128), jnp.float32),
        scratch_shapes=[pltpu.VMEM((NG, 128), jnp.float32)],
    )


def kernel(x, edge_index, batch, edge_attr, params):
    f32 = jnp.float32
    src = jnp.concatenate([edge_index[0], jnp.zeros((EPAD - E,), jnp.int32)])
    dst = jnp.concatenate([edge_index[1], jnp.full((EPAD - E,), N, jnp.int32)])
    attr = jnp.concatenate([edge_attr[:, 0], jnp.zeros((EPAD - E,), f32)])
    batch_p = jnp.concatenate(
        [batch, jnp.full((NPAD - N,), NG, jnp.int32)]).reshape(GRID, 1, BR)

    h = jnp.zeros((NPAD, 16), f32).at[:N, :IN_C].set(x)
    for li, lp in enumerate(params["layers"]):
        D = 16 if li == 0 else H
        w = lp["edge_lin"]["W"][0]
        b = lp["edge_lin"]["b"]
        W1 = lp["nn1"]["W"]
        if li == 0:
            w = jnp.pad(w, (0, 16 - IN_C))
            b = jnp.pad(b, (0, 16 - IN_C))
            W1 = jnp.pad(W1, ((0, 16 - IN_C), (0, 0)))
        wb = jnp.stack([w, b])
        agg = _sc_agg(D)(h, src, dst, attr, wb)
        aux = jnp.stack(
            [lp["nn1"]["b"], lp["nn2"]["b"], lp["bn_gamma"], lp["bn_beta"]])
        h = _mlp(D)(jnp.reshape(lp["eps"], (1,)), h, agg, W1, lp["nn2"]["W"], aux)

    wc2 = jnp.pad(params["cls"]["l2"]["W"], ((0, 0), (0, 128 - OUT_C)))
    bc2 = jnp.pad(params["cls"]["l2"]["b"], (0, 128 - OUT_C)).reshape(1, 128)
    out = _pool()(h, batch_p, params["cls"]["l1"]["W"],
                  params["cls"]["l1"]["b"].reshape(1, H), wc2, bc2)
    return out[:, :OUT_C]
